# baseline (speedup 1.0000x reference)
import jax
import jax.numpy as jnp
from jax import lax
from jax.experimental import pallas as pl
from jax.experimental.pallas import tpu as pltpu
from jax.experimental.pallas import tpu_sc as plsc

N = 8192
A = 8
D = 256
NC = 2
NS = 16
NW = NC * NS
ROWS_PER_W = N // NW


def _sc_tile(e):
    mesh = plsc.VectorSubcoreMesh(core_axis_name="core",
                                  subcore_axis_name="subcore")

    @pl.kernel(
        out_type=jax.ShapeDtypeStruct((N, D), jnp.float32),
        mesh=mesh,
        scratch_types=[pltpu.VMEM((A, D), jnp.float32)],
    )
    def k(e_hbm, o_hbm, buf):
        wid = lax.axis_index("subcore") * NC + lax.axis_index("core")
        base = wid * ROWS_PER_W
        pltpu.sync_copy(e_hbm, buf)
        pltpu.sync_copy(buf, o_hbm.at[pl.ds(base, A)])
        size = A
        while size < ROWS_PER_W:
            pltpu.sync_copy(o_hbm.at[pl.ds(base, size)],
                            o_hbm.at[pl.ds(base + size, size)])
            size *= 2

    return k(e)


def kernel(x, E_relative_position):
    del x
    return _sc_tile(E_relative_position)


# speedup vs baseline: 9.6215x; 9.6215x over previous
import jax
import jax.numpy as jnp
from jax import lax
from jax.experimental import pallas as pl
from jax.experimental.pallas import tpu as pltpu
from jax.experimental.pallas import tpu_sc as plsc

N = 8192
A = 8
D = 256
NC = 2
NS = 16
NW = NC * NS
ROWS_PER_W = N // NW


def _sc_tile(e):
    mesh = plsc.VectorSubcoreMesh(core_axis_name="core",
                                  subcore_axis_name="subcore")

    @pl.kernel(
        out_type=jax.ShapeDtypeStruct((N, D), jnp.float32),
        mesh=mesh,
        scratch_types=[pltpu.VMEM((2 * A, D), jnp.float32),
                       pltpu.VMEM_SHARED((ROWS_PER_W, D), jnp.float32)],
    )
    def k(e_hbm, o_hbm, buf, shared):
        sid = lax.axis_index("subcore")
        wid = sid * NC + lax.axis_index("core")
        pltpu.sync_copy(e_hbm, buf.at[pl.ds(0, A)])
        pltpu.sync_copy(e_hbm, buf.at[pl.ds(A, A)])
        pltpu.sync_copy(buf, shared.at[pl.ds(sid * 2 * A, 2 * A)])
        plsc.subcore_barrier()
        pltpu.sync_copy(shared, o_hbm.at[pl.ds(wid * ROWS_PER_W, ROWS_PER_W)])

    return k(e)


def kernel(x, E_relative_position):
    del x
    return _sc_tile(E_relative_position)
